# Initial kernel scaffold; baseline (speedup 1.0000x reference)
import jax
import jax.numpy as jnp
from jax.experimental import pallas as pl


def kernel(adj_matrix, W1, b1, W2, b2, Wfc, bfc):
    raise NotImplementedError("write your pallas kernel here")


# speedup vs baseline: 2616.4371x; 2616.4371x over previous
import jax
import jax.numpy as jnp
from jax.experimental import pallas as pl
from jax.experimental.pallas import tpu as pltpu

_N = 2048
_H = 32
_R = 256
_C = _N // _R
_PREC = jax.lax.Precision.HIGHEST
_DN_T = (((0,), (0,)), ((), ()))


def _at_dot(a_chunk, x_chunk):
    return jax.lax.dot_general(a_chunk, x_chunk, _DN_T, precision=_PREC,
                               preferred_element_type=jnp.float32)


def _gnn_kernel(adj_ref, w1_ref, b1_ref, w2_ref, b2_ref, wfc_ref, bfc_ref,
                out_ref, d_ref, t_ref, y2_ref, z_ref):
    def chunk(k):
        return (adj_ref[pl.ds(k * _R, _R), :] > 0).astype(jnp.float32)

    ones_col = jnp.ones((_R, 1), jnp.float32)

    d_ref[...] = jnp.zeros((_N, 1), jnp.float32)

    def p1(k, carry):
        d_ref[...] += _at_dot(chunk(k), ones_col)
        return carry

    jax.lax.fori_loop(0, _C, p1, 0)
    d_ref[...] = 1.0 / jnp.sqrt(d_ref[...] + 1.0)

    t_ref[...] = jnp.zeros((_N, 1), jnp.float32)

    def p2(k, carry):
        t_ref[...] += _at_dot(chunk(k), d_ref[pl.ds(k * _R, _R), :])
        return carry

    jax.lax.fori_loop(0, _C, p2, 0)

    dinv = d_ref[...]
    alpha = dinv * (t_ref[...] + dinv)
    x1 = jax.nn.relu(alpha * w1_ref[...] + b1_ref[...])
    y2_ref[...] = dinv * jnp.dot(x1, w2_ref[...], precision=_PREC,
                                 preferred_element_type=jnp.float32)

    z_ref[...] = jnp.zeros((_N, _H), jnp.float32)

    def p3(k, carry):
        z_ref[...] += _at_dot(chunk(k), y2_ref[pl.ds(k * _R, _R), :])
        return carry

    jax.lax.fori_loop(0, _C, p3, 0)

    x2 = jax.nn.relu(dinv * (z_ref[...] + y2_ref[...]) + b2_ref[...])
    pooled = jnp.sum(x2, axis=0, keepdims=True)
    out_ref[...] = jnp.dot(pooled, wfc_ref[...], precision=_PREC,
                           preferred_element_type=jnp.float32) + bfc_ref[...]


def kernel(adj_matrix, W1, b1, W2, b2, Wfc, bfc):
    return pl.pallas_call(
        _gnn_kernel,
        out_shape=jax.ShapeDtypeStruct((1, Wfc.shape[1]), jnp.float32),
        scratch_shapes=[
            pltpu.VMEM((_N, 1), jnp.float32),
            pltpu.VMEM((_N, 1), jnp.float32),
            pltpu.VMEM((_N, _H), jnp.float32),
            pltpu.VMEM((_N, _H), jnp.float32),
        ],
    )(adj_matrix, W1, b1.reshape(1, -1), W2, b2.reshape(1, -1), Wfc,
      bfc.reshape(1, -1))


# speedup vs baseline: 7177.6143x; 2.7433x over previous
import jax
import jax.numpy as jnp
from jax.experimental import pallas as pl
from jax.experimental.pallas import tpu as pltpu

_N = 2048
_H = 32
_R = 256
_C = _N // _R
_PREC = jax.lax.Precision.HIGHEST
_DN_T = (((0,), (0,)), ((), ()))


def _gnn_kernel(adj_ref, w1_ref, b1_ref, w2_ref, b2_ref, wfc_ref, bfc_ref,
                out_ref, row_ref, dcol_ref, y2_ref, z_ref, y2h_ref, y2l_ref):
    def fchunk(k):
        return (adj_ref[pl.ds(k * _R, _R), :] > 0).astype(jnp.float32)

    row_ref[...] = jnp.zeros((1, _N), jnp.float32)

    def p1(k, carry):
        row_ref[...] += jnp.sum(fchunk(k), axis=0, keepdims=True)
        return carry

    jax.lax.fori_loop(0, _C, p1, 0)
    dinv_row = 1.0 / jnp.sqrt(row_ref[...] + 1.0)
    dcol_ref[...] = jnp.reshape(dinv_row, (_N, 1))

    row_ref[...] = jnp.zeros((1, _N), jnp.float32)

    def p2(k, carry):
        d = dcol_ref[pl.ds(k * _R, _R), :]
        row_ref[...] += jnp.sum(fchunk(k) * d, axis=0, keepdims=True)
        return carry

    jax.lax.fori_loop(0, _C, p2, 0)

    alpha_row = dinv_row * (row_ref[...] + dinv_row)
    alpha = jnp.reshape(alpha_row, (_N, 1))
    dinv = dcol_ref[...]
    x1 = jax.nn.relu(alpha * w1_ref[...] + b1_ref[...])
    y2 = dinv * jnp.dot(x1, w2_ref[...], precision=_PREC,
                        preferred_element_type=jnp.float32)
    y2_ref[...] = y2

    y2_hi = y2.astype(jnp.bfloat16)
    y2h_ref[...] = y2_hi
    y2l_ref[...] = (y2 - y2_hi.astype(jnp.float32)).astype(jnp.bfloat16)
    z_ref[...] = jnp.zeros((_N, _H), jnp.float32)

    def p3(k, carry):
        a_bf = (adj_ref[pl.ds(k * _R, _R), :] > 0).astype(jnp.bfloat16)
        hi = jax.lax.dot_general(a_bf, y2h_ref[pl.ds(k * _R, _R), :], _DN_T,
                                 preferred_element_type=jnp.float32)
        lo = jax.lax.dot_general(a_bf, y2l_ref[pl.ds(k * _R, _R), :], _DN_T,
                                 preferred_element_type=jnp.float32)
        z_ref[...] += hi + lo
        return carry

    jax.lax.fori_loop(0, _C, p3, 0)

    x2 = jax.nn.relu(dinv * (z_ref[...] + y2_ref[...]) + b2_ref[...])
    pooled = jnp.sum(x2, axis=0, keepdims=True)
    out_ref[...] = jnp.dot(pooled, wfc_ref[...], precision=_PREC,
                           preferred_element_type=jnp.float32) + bfc_ref[...]


def kernel(adj_matrix, W1, b1, W2, b2, Wfc, bfc):
    return pl.pallas_call(
        _gnn_kernel,
        out_shape=jax.ShapeDtypeStruct((1, Wfc.shape[1]), jnp.float32),
        scratch_shapes=[
            pltpu.VMEM((1, _N), jnp.float32),
            pltpu.VMEM((_N, 1), jnp.float32),
            pltpu.VMEM((_N, _H), jnp.float32),
            pltpu.VMEM((_N, _H), jnp.float32),
            pltpu.VMEM((_N, _H), jnp.bfloat16),
            pltpu.VMEM((_N, _H), jnp.bfloat16),
        ],
    )(adj_matrix, W1, b1.reshape(1, -1), W2, b2.reshape(1, -1), Wfc,
      bfc.reshape(1, -1))


# speedup vs baseline: 7852.4676x; 1.0940x over previous
import jax
import jax.numpy as jnp
from jax.experimental import pallas as pl
from jax.experimental.pallas import tpu as pltpu

_N = 2048
_H = 32
_R = 256
_C = _N // _R
_PREC = jax.lax.Precision.HIGHEST
_DN_T = (((0,), (0,)), ((), ()))


def _gnn_kernel(adj_ref, w1_ref, b1_ref, w2_ref, b2_ref, wfc_ref, bfc_ref,
                out_ref, row_ref, dcol_ref, y2_ref, z_ref, ycat_ref):
    def fchunk(k):
        return (adj_ref[pl.ds(k * _R, _R), :] > 0).astype(jnp.float32)

    row_ref[...] = jnp.zeros((1, _N), jnp.float32)

    def p1(k, carry):
        row_ref[...] += jnp.sum(fchunk(k), axis=0, keepdims=True)
        return carry

    jax.lax.fori_loop(0, _C, p1, 0)
    dinv_row = 1.0 / jnp.sqrt(row_ref[...] + 1.0)
    dcol_ref[...] = jnp.reshape(dinv_row, (_N, 1))

    row_ref[...] = jnp.zeros((1, _N), jnp.float32)

    def p2(k, carry):
        d = dcol_ref[pl.ds(k * _R, _R), :]
        row_ref[...] += jnp.sum(fchunk(k) * d, axis=0, keepdims=True)
        return carry

    jax.lax.fori_loop(0, _C, p2, 0)

    alpha_row = dinv_row * (row_ref[...] + dinv_row)
    alpha = jnp.reshape(alpha_row, (_N, 1))
    dinv = dcol_ref[...]
    x1 = jax.nn.relu(alpha * w1_ref[...] + b1_ref[...])
    y2 = dinv * jnp.dot(x1, w2_ref[...], precision=_PREC,
                        preferred_element_type=jnp.float32)
    y2_ref[...] = y2

    y2_hi = y2.astype(jnp.bfloat16)
    ycat_ref[...] = jnp.concatenate(
        [y2_hi, (y2 - y2_hi.astype(jnp.float32)).astype(jnp.bfloat16)],
        axis=1)
    z_ref[...] = jnp.zeros((_N, 2 * _H), jnp.float32)

    def p3(k, carry):
        a_bf = (adj_ref[pl.ds(k * _R, _R), :] > 0).astype(jnp.bfloat16)
        z_ref[...] += jax.lax.dot_general(
            a_bf, ycat_ref[pl.ds(k * _R, _R), :], _DN_T,
            preferred_element_type=jnp.float32)
        return carry

    jax.lax.fori_loop(0, _C, p3, 0)

    z = z_ref[:, :_H] + z_ref[:, _H:]
    x2 = jax.nn.relu(dinv * (z + y2_ref[...]) + b2_ref[...])
    pooled = jnp.sum(x2, axis=0, keepdims=True)
    out_ref[...] = jnp.dot(pooled, wfc_ref[...], precision=_PREC,
                           preferred_element_type=jnp.float32) + bfc_ref[...]


def kernel(adj_matrix, W1, b1, W2, b2, Wfc, bfc):
    return pl.pallas_call(
        _gnn_kernel,
        out_shape=jax.ShapeDtypeStruct((1, Wfc.shape[1]), jnp.float32),
        scratch_shapes=[
            pltpu.VMEM((1, _N), jnp.float32),
            pltpu.VMEM((_N, 1), jnp.float32),
            pltpu.VMEM((_N, _H), jnp.float32),
            pltpu.VMEM((_N, 2 * _H), jnp.float32),
            pltpu.VMEM((_N, 2 * _H), jnp.bfloat16),
        ],
    )(adj_matrix, W1, b1.reshape(1, -1), W2, b2.reshape(1, -1), Wfc,
      bfc.reshape(1, -1))


# speedup vs baseline: 8013.1585x; 1.0205x over previous
import jax
import jax.numpy as jnp
from jax.experimental import pallas as pl
from jax.experimental.pallas import tpu as pltpu

_N = 2048
_H = 32
_R = 256
_C = _N // _R
_PREC = jax.lax.Precision.HIGHEST
_DN_T = (((0,), (0,)), ((), ()))


def _gnn_kernel(adj_ref, w1_ref, b1_ref, w2_ref, b2_ref, wfc_ref, bfc_ref,
                out_ref, row_ref, dcol_ref, y2_ref, z_ref, ycat_ref, abf_ref):
    row_ref[...] = jnp.zeros((1, _N), jnp.float32)

    def p1(k, carry):
        af = (adj_ref[pl.ds(k * _R, _R), :] > 0).astype(jnp.float32)
        abf_ref[pl.ds(k * _R, _R), :] = af.astype(jnp.bfloat16)
        row_ref[...] += jnp.sum(af, axis=0, keepdims=True)
        return carry

    jax.lax.fori_loop(0, _C, p1, 0)
    dinv_row = 1.0 / jnp.sqrt(row_ref[...] + 1.0)
    dcol_ref[...] = jnp.reshape(dinv_row, (_N, 1))

    row_ref[...] = jnp.zeros((1, _N), jnp.float32)

    def p2(k, carry):
        d = dcol_ref[pl.ds(k * _R, _R), :]
        af = abf_ref[pl.ds(k * _R, _R), :].astype(jnp.float32)
        row_ref[...] += jnp.sum(af * d, axis=0, keepdims=True)
        return carry

    jax.lax.fori_loop(0, _C, p2, 0)

    alpha_row = dinv_row * (row_ref[...] + dinv_row)
    alpha = jnp.reshape(alpha_row, (_N, 1))
    dinv = dcol_ref[...]
    x1 = jax.nn.relu(alpha * w1_ref[...] + b1_ref[...])
    y2 = dinv * jnp.dot(x1, w2_ref[...], precision=_PREC,
                        preferred_element_type=jnp.float32)
    y2_ref[...] = y2

    y2_hi = y2.astype(jnp.bfloat16)
    ycat_ref[...] = jnp.concatenate(
        [y2_hi, (y2 - y2_hi.astype(jnp.float32)).astype(jnp.bfloat16)],
        axis=1)
    z_ref[...] = jnp.zeros((_N, 2 * _H), jnp.float32)

    def p3(k, carry):
        z_ref[...] += jax.lax.dot_general(
            abf_ref[pl.ds(k * _R, _R), :], ycat_ref[pl.ds(k * _R, _R), :],
            _DN_T, preferred_element_type=jnp.float32)
        return carry

    jax.lax.fori_loop(0, _C, p3, 0)

    z = z_ref[:, :_H] + z_ref[:, _H:]
    x2 = jax.nn.relu(dinv * (z + y2_ref[...]) + b2_ref[...])
    pooled = jnp.sum(x2, axis=0, keepdims=True)
    out_ref[...] = jnp.dot(pooled, wfc_ref[...], precision=_PREC,
                           preferred_element_type=jnp.float32) + bfc_ref[...]


def kernel(adj_matrix, W1, b1, W2, b2, Wfc, bfc):
    return pl.pallas_call(
        _gnn_kernel,
        out_shape=jax.ShapeDtypeStruct((1, Wfc.shape[1]), jnp.float32),
        scratch_shapes=[
            pltpu.VMEM((1, _N), jnp.float32),
            pltpu.VMEM((_N, 1), jnp.float32),
            pltpu.VMEM((_N, _H), jnp.float32),
            pltpu.VMEM((_N, 2 * _H), jnp.float32),
            pltpu.VMEM((_N, 2 * _H), jnp.bfloat16),
            pltpu.VMEM((_N, _N), jnp.bfloat16),
        ],
    )(adj_matrix, W1, b1.reshape(1, -1), W2, b2.reshape(1, -1), Wfc,
      bfc.reshape(1, -1))
